# baseline (speedup 1.0000x reference)
import jax
import jax.numpy as jnp
from jax.experimental import pallas as pl
from jax.experimental.pallas import tpu as pltpu

_N = 10000
_NFEAT = 128
_NHID = 128
_DHID1 = 64

_BI = 80
_BD = 200


def _dot(a, b):
    return jnp.dot(a, b, preferred_element_type=jnp.float32)


def _fused_kernel(ap_ref, an_ref, x_ref, wp_ref, wn_ref, bp_ref, bn_ref,
                  w1p_ref, w1n_ref, b1_ref, w2_ref, b2_ref, w3_ref, b3_ref,
                  wd_ref, y_ref, xwp_ref, xwn_ref, h_ref, t_ref):
    i = pl.program_id(0)
    n_enc = _N // _BI

    @pl.when(i == 0)
    def _():
        x = x_ref[...]
        xwp_ref[...] = _dot(x, wp_ref[...])
        xwn_ref[...] = _dot(x, wn_ref[...])

    @pl.when(i < n_enc)
    def _():
        zp = jax.nn.relu(_dot(ap_ref[...], xwp_ref[...]) + bp_ref[...])
        zn = jax.nn.relu(_dot(an_ref[...], xwn_ref[...]) + bn_ref[...])
        h1 = jax.nn.relu(_dot(zp, w1p_ref[...]) + _dot(zn, w1n_ref[...])
                         + b1_ref[...])
        h2 = jax.nn.relu(_dot(h1, w2_ref[...]) + b2_ref[...])
        h = _dot(h2, w3_ref[...]) + b3_ref[...]
        h_ref[pl.ds(i * _BI, _BI), :] = h
        t_ref[pl.ds(i * _BI, _BI), :] = _dot(h, wd_ref[...])

    @pl.when(i >= n_enc)
    def _():
        j = i - n_enc
        t_blk = t_ref[pl.ds(j * _BD, _BD), :]
        y_ref[...] = jax.lax.dot_general(
            t_blk, h_ref[...], (((1,), (1,)), ((), ())),
            preferred_element_type=jnp.float32)


def kernel(x, adj_norm_pos, adj_norm_neg, W_pos, b_pos, W_neg, b_neg,
           W1, b1, W2, b2, W3, b3, Wd):
    f32 = jnp.float32
    n_enc = _N // _BI
    n_dec = _N // _BD
    a_idx = lambda i: (jnp.minimum(i, n_enc - 1), 0)
    full = lambda shape: pl.BlockSpec(shape, lambda i: (0, 0))

    y = pl.pallas_call(
        _fused_kernel,
        grid=(n_enc + n_dec,),
        in_specs=[
            pl.BlockSpec((_BI, _N), a_idx),
            pl.BlockSpec((_BI, _N), a_idx),
            full((_N, _NFEAT)),
            full((_NFEAT, _NHID)),
            full((_NFEAT, _NHID)),
            full((1, _NHID)),
            full((1, _NHID)),
            full((_NHID, _DHID1)),
            full((_NHID, _DHID1)),
            full((1, _DHID1)),
            full((_DHID1, 2 * _DHID1)),
            full((1, 2 * _DHID1)),
            full((2 * _DHID1, _DHID1)),
            full((1, _DHID1)),
            full((_DHID1, _DHID1)),
        ],
        out_specs=pl.BlockSpec((_BD, _N), lambda i: (jnp.maximum(i - n_enc, 0), 0)),
        out_shape=jax.ShapeDtypeStruct((_N, _N), f32),
        scratch_shapes=[
            pltpu.VMEM((_N, _NHID), f32),
            pltpu.VMEM((_N, _NHID), f32),
            pltpu.VMEM((_N, _DHID1), f32),
            pltpu.VMEM((_N, _DHID1), f32),
        ],
    )(adj_norm_pos, adj_norm_neg, x, W_pos, W_neg,
      b_pos.reshape(1, -1), b_neg.reshape(1, -1),
      W1[:_NHID], W1[_NHID:], b1.reshape(1, -1),
      W2, b2.reshape(1, -1), W3, b3.reshape(1, -1), Wd)
    return y


# speedup vs baseline: 1.1064x; 1.1064x over previous
import jax
import jax.numpy as jnp
from jax.experimental import pallas as pl
from jax.experimental.pallas import tpu as pltpu

_N = 10000
_NFEAT = 128
_NHID = 128
_DHID1 = 64

_BI = 200
_BD = 200


def _dot(a, b):
    return jnp.dot(a, b, preferred_element_type=jnp.float32)


def _xw_kernel(x_ref, wp_ref, wn_ref, xwp_ref, xwn_ref):
    x = x_ref[...]
    xwp_ref[...] = _dot(x, wp_ref[...]).astype(jnp.bfloat16)
    xwn_ref[...] = _dot(x, wn_ref[...]).astype(jnp.bfloat16)


def _fused_kernel(ap_ref, an_ref, xwp_ref, xwn_ref, bp_ref, bn_ref,
                  w1p_ref, w1n_ref, b1_ref, w2_ref, b2_ref, w3_ref, b3_ref,
                  wd_ref, y_ref, h_ref, t_ref):
    i = pl.program_id(0)
    n_enc = _N // _BI

    @pl.when(i < n_enc)
    def _():
        xwp = xwp_ref[...].astype(jnp.float32)
        xwn = xwn_ref[...].astype(jnp.float32)
        zp = jax.nn.relu(_dot(ap_ref[...], xwp) + bp_ref[...])
        zn = jax.nn.relu(_dot(an_ref[...], xwn) + bn_ref[...])
        h1 = jax.nn.relu(_dot(zp, w1p_ref[...]) + _dot(zn, w1n_ref[...])
                         + b1_ref[...])
        h2 = jax.nn.relu(_dot(h1, w2_ref[...]) + b2_ref[...])
        h = _dot(h2, w3_ref[...]) + b3_ref[...]
        h_ref[pl.ds(i * _BI, _BI), :] = h.astype(jnp.bfloat16)
        t_ref[pl.ds(i * _BI, _BI), :] = _dot(h, wd_ref[...]).astype(jnp.bfloat16)

    @pl.when(i >= n_enc)
    def _():
        j = i - n_enc
        t_blk = t_ref[pl.ds(j * _BD, _BD), :]
        y_ref[...] = jax.lax.dot_general(
            t_blk, h_ref[...], (((1,), (1,)), ((), ())),
            preferred_element_type=jnp.float32)


def kernel(x, adj_norm_pos, adj_norm_neg, W_pos, b_pos, W_neg, b_neg,
           W1, b1, W2, b2, W3, b3, Wd):
    f32 = jnp.float32
    bf16 = jnp.bfloat16
    n_enc = _N // _BI
    n_dec = _N // _BD
    a_idx = lambda i: (jnp.minimum(i, n_enc - 1), 0)
    full = lambda shape: pl.BlockSpec(shape, lambda i: (0, 0))

    xwp, xwn = pl.pallas_call(
        _xw_kernel,
        out_shape=[jax.ShapeDtypeStruct((_N, _NHID), bf16)] * 2,
    )(x, W_pos, W_neg)

    y = pl.pallas_call(
        _fused_kernel,
        grid=(n_enc + n_dec,),
        in_specs=[
            pl.BlockSpec((_BI, _N), a_idx),
            pl.BlockSpec((_BI, _N), a_idx),
            full((_N, _NHID)),
            full((_N, _NHID)),
            full((1, _NHID)),
            full((1, _NHID)),
            full((_NHID, _DHID1)),
            full((_NHID, _DHID1)),
            full((1, _DHID1)),
            full((_DHID1, 2 * _DHID1)),
            full((1, 2 * _DHID1)),
            full((2 * _DHID1, _DHID1)),
            full((1, _DHID1)),
            full((_DHID1, _DHID1)),
        ],
        out_specs=pl.BlockSpec((_BD, _N), lambda i: (jnp.maximum(i - n_enc, 0), 0)),
        out_shape=jax.ShapeDtypeStruct((_N, _N), f32),
        scratch_shapes=[
            pltpu.VMEM((_N, _DHID1), bf16),
            pltpu.VMEM((_N, _DHID1), bf16),
        ],
    )(adj_norm_pos, adj_norm_neg, xwp, xwn,
      b_pos.reshape(1, -1), b_neg.reshape(1, -1),
      W1[:_NHID], W1[_NHID:], b1.reshape(1, -1),
      W2, b2.reshape(1, -1), W3, b3.reshape(1, -1), Wd)
    return y


# speedup vs baseline: 1.1134x; 1.0063x over previous
import jax
import jax.numpy as jnp
from jax.experimental import pallas as pl
from jax.experimental.pallas import tpu as pltpu

_N = 10000
_NFEAT = 128
_NHID = 128
_DHID1 = 64

_BI = 200
_BD = 200


def _dot(a, b):
    return jnp.dot(a, b, preferred_element_type=jnp.float32)


def _fused_kernel(ap_ref, an_ref, x_ref, wp_ref, wn_ref, bp_ref, bn_ref,
                  w1p_ref, w1n_ref, b1_ref, w2_ref, b2_ref, w3_ref, b3_ref,
                  wd_ref, y_ref, xwp_ref, xwn_ref, h_ref, t_ref):
    i = pl.program_id(0)
    n_enc = _N // _BI

    @pl.when(i == 0)
    def _():
        x = x_ref[...].astype(jnp.float32)
        xwp_ref[...] = _dot(x, wp_ref[...]).astype(jnp.bfloat16)
        xwn_ref[...] = _dot(x, wn_ref[...]).astype(jnp.bfloat16)

    @pl.when(i < n_enc)
    def _():
        xwp = xwp_ref[...].astype(jnp.float32)
        xwn = xwn_ref[...].astype(jnp.float32)
        zp = jax.nn.relu(_dot(ap_ref[...], xwp) + bp_ref[...])
        zn = jax.nn.relu(_dot(an_ref[...], xwn) + bn_ref[...])
        h1 = jax.nn.relu(_dot(zp, w1p_ref[...]) + _dot(zn, w1n_ref[...])
                         + b1_ref[...])
        h2 = jax.nn.relu(_dot(h1, w2_ref[...]) + b2_ref[...])
        h = _dot(h2, w3_ref[...]) + b3_ref[...]
        h_ref[pl.ds(i * _BI, _BI), :] = h.astype(jnp.bfloat16)
        t_ref[pl.ds(i * _BI, _BI), :] = _dot(h, wd_ref[...]).astype(jnp.bfloat16)

    @pl.when(i >= n_enc)
    def _():
        j = i - n_enc
        t_blk = t_ref[pl.ds(j * _BD, _BD), :]
        y_ref[...] = jax.lax.dot_general(
            t_blk, h_ref[...], (((1,), (1,)), ((), ())),
            preferred_element_type=jnp.float32)


def kernel(x, adj_norm_pos, adj_norm_neg, W_pos, b_pos, W_neg, b_neg,
           W1, b1, W2, b2, W3, b3, Wd):
    f32 = jnp.float32
    bf16 = jnp.bfloat16
    n_enc = _N // _BI
    n_dec = _N // _BD
    a_idx = lambda i: (jnp.minimum(i, n_enc - 1), 0)
    full = lambda shape: pl.BlockSpec(shape, lambda i: (0, 0))

    y = pl.pallas_call(
        _fused_kernel,
        grid=(n_enc + n_dec,),
        in_specs=[
            pl.BlockSpec((_BI, _N), a_idx),
            pl.BlockSpec((_BI, _N), a_idx),
            full((_N, _NFEAT)),
            full((_NFEAT, _NHID)),
            full((_NFEAT, _NHID)),
            full((1, _NHID)),
            full((1, _NHID)),
            full((_NHID, _DHID1)),
            full((_NHID, _DHID1)),
            full((1, _DHID1)),
            full((_DHID1, 2 * _DHID1)),
            full((1, 2 * _DHID1)),
            full((2 * _DHID1, _DHID1)),
            full((1, _DHID1)),
            full((_DHID1, _DHID1)),
        ],
        out_specs=pl.BlockSpec((_BD, _N), lambda i: (jnp.maximum(i - n_enc, 0), 0)),
        out_shape=jax.ShapeDtypeStruct((_N, _N), f32),
        compiler_params=pltpu.CompilerParams(vmem_limit_bytes=62 * 1024 * 1024),
        scratch_shapes=[
            pltpu.VMEM((_N, _NHID), bf16),
            pltpu.VMEM((_N, _NHID), bf16),
            pltpu.VMEM((_N, _DHID1), bf16),
            pltpu.VMEM((_N, _DHID1), bf16),
        ],
    )(adj_norm_pos, adj_norm_neg, x.astype(bf16), W_pos, W_neg,
      b_pos.reshape(1, -1), b_neg.reshape(1, -1),
      W1[:_NHID], W1[_NHID:], b1.reshape(1, -1),
      W2, b2.reshape(1, -1), W3, b3.reshape(1, -1), Wd)
    return y
